# Initial kernel scaffold; baseline (speedup 1.0000x reference)
import jax
import jax.numpy as jnp
from jax.experimental import pallas as pl


def kernel(input_ids, embedding_weight):
    raise NotImplementedError("write your pallas kernel here")


# speedup vs baseline: 1.2740x; 1.2740x over previous
import functools

import jax
import jax.numpy as jnp
from jax import lax
from jax.experimental import pallas as pl
from jax.experimental.pallas import tpu as pltpu
from jax.experimental.pallas import tpu_sc as plsc

_LIST_OF_N = [100000] * 8
_D = 128
_B, _S = 4, 2048
_H = len(_LIST_OF_N)
_TOTAL = _B * _S * _H

_CHUNK = 128


def _make_gather():
    info = plsc.get_sparse_core_info()
    nc, ns, nl = info.num_cores, info.num_subcores, info.num_lanes
    nw = nc * ns
    per_w = _TOTAL // nw
    n_chunks = per_w // _CHUNK

    mesh = plsc.VectorSubcoreMesh(core_axis_name="c", subcore_axis_name="s")

    @functools.partial(
        pl.kernel,
        mesh=mesh,
        out_type=jax.ShapeDtypeStruct((_TOTAL, _D), jnp.float32),
        scratch_types=[
            pltpu.VMEM((per_w,), jnp.int32),
            pltpu.VMEM((_CHUNK,), jnp.int32),
            pltpu.VMEM((_CHUNK, _D), jnp.float32),
            pltpu.SemaphoreType.DMA,
        ],
    )
    def gather_kernel(table_hbm, idx_hbm, out_hbm, idx_v, idxc_v, rows_v, sem):
        wid = lax.axis_index("s") * nc + lax.axis_index("c")
        base = wid * per_w

        pltpu.sync_copy(idx_hbm.at[pl.ds(base, per_w)], idx_v)

        off = (lax.iota(jnp.int32, nl) & 7) * 100000

        def chunk_body(c, carry):
            cbase = c * _CHUNK
            for j in range(_CHUNK // nl):
                sl = pl.ds(cbase + j * nl, nl)
                idxc_v[pl.ds(j * nl, nl)] = idx_v[sl] + off
            pltpu.async_copy(table_hbm.at[idxc_v], rows_v, sem).wait()
            pltpu.sync_copy(rows_v, out_hbm.at[pl.ds(base + cbase, _CHUNK)])
            return carry

        lax.fori_loop(0, n_chunks, chunk_body, 0)

    return gather_kernel


_gather = _make_gather()


@jax.jit
def kernel(input_ids, embedding_weight):
    flat_ids = input_ids.reshape(_TOTAL).astype(jnp.int32)
    out = _gather(embedding_weight, flat_ids)
    return out.reshape(_B, _S, _H, _D)


# speedup vs baseline: 1.5976x; 1.2541x over previous
import functools

import jax
import jax.numpy as jnp
from jax import lax
from jax.experimental import pallas as pl
from jax.experimental.pallas import tpu as pltpu
from jax.experimental.pallas import tpu_sc as plsc

_LIST_OF_N = [100000] * 8
_D = 128
_B, _S = 4, 2048
_H = len(_LIST_OF_N)
_TOTAL = _B * _S * _H

_CHUNK = 128
_NSLOT = 6
_LOOKAHEAD = 3


def _make_gather():
    info = plsc.get_sparse_core_info()
    nc, ns, nl = info.num_cores, info.num_subcores, info.num_lanes
    nw = nc * ns
    per_w = _TOTAL // nw
    n_chunks = per_w // _CHUNK

    mesh = plsc.VectorSubcoreMesh(core_axis_name="c", subcore_axis_name="s")

    @functools.partial(
        pl.kernel,
        mesh=mesh,
        out_type=jax.ShapeDtypeStruct((_TOTAL, _D), jnp.float32),
        scratch_types=[
            pltpu.VMEM((per_w,), jnp.int32),
            pltpu.VMEM((_NSLOT, _CHUNK, _D), jnp.float32),
        ] + [pltpu.SemaphoreType.DMA] * _NSLOT,
    )
    def gather_kernel(table_hbm, idx_hbm, out_hbm, idx_v, rows_v, *sems):
        wid = lax.axis_index("s") * nc + lax.axis_index("c")
        base = wid * per_w

        pltpu.sync_copy(idx_hbm.at[pl.ds(base, per_w)], idx_v)

        off = (lax.iota(jnp.int32, nl) & 7) * 100000

        gathers = {}
        outs = {}
        for t in range(n_chunks + _LOOKAHEAD):
            c = t
            if c < n_chunks:
                b = c % _NSLOT
                if c >= _NSLOT:
                    outs[b].wait()
                for j in range(_CHUNK // nl):
                    sl = pl.ds(c * _CHUNK + j * nl, nl)
                    idx_v[sl] = idx_v[sl] + off
                gathers[b] = pltpu.async_copy(
                    table_hbm.at[idx_v.at[pl.ds(c * _CHUNK, _CHUNK)]],
                    rows_v.at[b], sems[b])
            d = t - _LOOKAHEAD
            if 0 <= d < n_chunks:
                bd = d % _NSLOT
                gathers[bd].wait()
                outs[bd] = pltpu.async_copy(
                    rows_v.at[bd],
                    out_hbm.at[pl.ds(base + d * _CHUNK, _CHUNK)], sems[bd])
        for d in range(n_chunks - _NSLOT, n_chunks):
            outs[d % _NSLOT].wait()

    return gather_kernel


_gather = _make_gather()


@jax.jit
def kernel(input_ids, embedding_weight):
    flat_ids = input_ids.reshape(_TOTAL).astype(jnp.int32)
    out = _gather(embedding_weight, flat_ids)
    return out.reshape(_B, _S, _H, _D)


# speedup vs baseline: 1.6030x; 1.0034x over previous
import functools

import jax
import jax.numpy as jnp
from jax import lax
from jax.experimental import pallas as pl
from jax.experimental.pallas import tpu as pltpu
from jax.experimental.pallas import tpu_sc as plsc

_LIST_OF_N = [100000] * 8
_D = 128
_B, _S = 4, 2048
_H = len(_LIST_OF_N)
_TOTAL = _B * _S * _H

_CHUNK = 128
_NSLOT = 6
_LOOKAHEAD = 3


def _make_gather():
    info = plsc.get_sparse_core_info()
    nc, ns, nl = info.num_cores, info.num_subcores, info.num_lanes
    nw = nc * ns
    per_w = _TOTAL // nw
    n_chunks = per_w // _CHUNK

    mesh = plsc.VectorSubcoreMesh(core_axis_name="c", subcore_axis_name="s")

    @functools.partial(
        pl.kernel,
        mesh=mesh,
        out_type=jax.ShapeDtypeStruct((_B, _S, _H, _D), jnp.float32),
        scratch_types=[
            pltpu.VMEM((per_w // _CHUNK, _CHUNK), jnp.int32),
            pltpu.VMEM((_NSLOT, _CHUNK, _D), jnp.float32),
        ] + [pltpu.SemaphoreType.DMA] * _NSLOT,
    )
    def gather_kernel(table_hbm, ids_hbm, out_hbm, idx_v, rows_v, *sems):
        wid = lax.axis_index("s") * nc + lax.axis_index("c")
        base = wid * per_w

        idx_hbm = ids_hbm
        outf_hbm = out_hbm.reshape(_TOTAL, _D)

        pltpu.sync_copy(
            idx_hbm.at[pl.ds(wid * (per_w // _CHUNK), per_w // _CHUNK)], idx_v)

        off = (lax.iota(jnp.int32, nl) & 7) * 100000

        gathers = {}
        outs = {}
        for t in range(n_chunks + _LOOKAHEAD):
            c = t
            if c < n_chunks:
                b = c % _NSLOT
                if c >= _NSLOT:
                    outs[b].wait()
                for j in range(_CHUNK // nl):
                    sl = pl.ds(j * nl, nl)
                    idx_v[c, sl] = idx_v[c, sl] + off
                gathers[b] = pltpu.async_copy(
                    table_hbm.at[idx_v.at[c]],
                    rows_v.at[b], sems[b])
            d = t - _LOOKAHEAD
            if 0 <= d < n_chunks:
                bd = d % _NSLOT
                gathers[bd].wait()
                outs[bd] = pltpu.async_copy(
                    rows_v.at[bd],
                    outf_hbm.at[pl.ds(base + d * _CHUNK, _CHUNK)], sems[bd])
        for d in range(n_chunks - _NSLOT, n_chunks):
            outs[d % _NSLOT].wait()

    return gather_kernel


_gather = _make_gather()


@jax.jit
def kernel(input_ids, embedding_weight):
    flat_ids = input_ids.reshape(_TOTAL // _CHUNK, _CHUNK).astype(jnp.int32)
    return _gather(embedding_weight, flat_ids)


# speedup vs baseline: 1.6409x; 1.0237x over previous
import functools

import jax
import jax.numpy as jnp
from jax import lax
from jax.experimental import pallas as pl
from jax.experimental.pallas import tpu as pltpu
from jax.experimental.pallas import tpu_sc as plsc

_LIST_OF_N = [100000] * 8
_D = 128
_B, _S = 4, 2048
_H = len(_LIST_OF_N)
_TOTAL = _B * _S * _H

_CHUNK = 128
_NSLOT = 7
_LOOKAHEAD = 5


def _make_gather():
    info = plsc.get_sparse_core_info()
    nc, ns, nl = info.num_cores, info.num_subcores, info.num_lanes
    nw = nc * ns
    per_w = _TOTAL // nw
    n_chunks = per_w // _CHUNK

    mesh = plsc.VectorSubcoreMesh(core_axis_name="c", subcore_axis_name="s")

    @functools.partial(
        pl.kernel,
        mesh=mesh,
        out_type=jax.ShapeDtypeStruct((_B, _S, _H, _D), jnp.float32),
        scratch_types=[
            pltpu.VMEM((per_w // _CHUNK, _CHUNK), jnp.int32),
            pltpu.VMEM((_NSLOT, _CHUNK, _D), jnp.float32),
        ] + [pltpu.SemaphoreType.DMA] * _NSLOT,
    )
    def gather_kernel(table_hbm, ids_hbm, out_hbm, idx_v, rows_v, *sems):
        wid = lax.axis_index("s") * nc + lax.axis_index("c")
        base = wid * per_w

        idx_hbm = ids_hbm
        outf_hbm = out_hbm.reshape(_TOTAL, _D)

        pltpu.sync_copy(
            idx_hbm.at[pl.ds(wid * (per_w // _CHUNK), per_w // _CHUNK)], idx_v)

        off = (lax.iota(jnp.int32, nl) & 7) * 100000

        gathers = {}
        outs = {}
        for t in range(n_chunks + _LOOKAHEAD):
            c = t
            if c < n_chunks:
                b = c % _NSLOT
                if c >= _NSLOT:
                    outs[b].wait()
                for j in range(_CHUNK // nl):
                    sl = pl.ds(j * nl, nl)
                    idx_v[c, sl] = idx_v[c, sl] + off
                gathers[b] = pltpu.async_copy(
                    table_hbm.at[idx_v.at[c]],
                    rows_v.at[b], sems[b])
            d = t - _LOOKAHEAD
            if 0 <= d < n_chunks:
                bd = d % _NSLOT
                gathers[bd].wait()
                outs[bd] = pltpu.async_copy(
                    rows_v.at[bd],
                    outf_hbm.at[pl.ds(base + d * _CHUNK, _CHUNK)], sems[bd])
        for d in range(n_chunks - _NSLOT, n_chunks):
            outs[d % _NSLOT].wait()

    return gather_kernel


_gather = _make_gather()


@jax.jit
def kernel(input_ids, embedding_weight):
    flat_ids = input_ids.reshape(_TOTAL // _CHUNK, _CHUNK).astype(jnp.int32)
    return _gather(embedding_weight, flat_ids)
